# baseline (speedup 1.0000x reference)
import jax
import jax.numpy as jnp
from jax import lax
from jax.experimental import pallas as pl
from jax.experimental.pallas import tpu as pltpu
from jax.experimental.pallas import tpu_sc as plsc

N_NODES = 10000
N_EDGES = 320000
N_GRAPHS = 100
IN_DIM = 128
HID_DIM = 64
OUT_DIM = 64

NT = 32
CH = 128
CPT = 80
EPT = CPT * CH
E_PAD = NT * EPT
NB = 10240
NBUF = 4
NRND = CPT // NBUF
N_AGG = 10112
RPS = N_AGG // 16

BLK = 400
NBLK = N_NODES // BLK

_MESH = plsc.VectorSubcoreMesh(core_axis_name="c", subcore_axis_name="s")

_SC_PARAMS = pltpu.CompilerParams()
if "needs_layout_passes" in pltpu.CompilerParams.__dataclass_fields__:
    import dataclasses as _dataclasses
    _SC_PARAMS = _dataclasses.replace(
        _SC_PARAMS, needs_layout_passes=False, use_tc_tiling_on_sc=False)


def _sc_degrees_body(src_hbm, dst_hbm, out_hbm, sidx, didx, cnt_s, cnt_d):
    c = lax.axis_index("c")
    s = lax.axis_index("s")
    t = c * 16 + s
    pltpu.sync_copy(src_hbm.at[t], sidx)
    pltpu.sync_copy(dst_hbm.at[t], didx)

    zero16 = jnp.zeros((16,), jnp.float32)

    @pl.loop(0, NB // 16)
    def _zero(i):
        cnt_s[pl.ds(i * 16, 16)] = zero16
        cnt_d[pl.ds(i * 16, 16)] = zero16

    ones16 = jnp.ones((16,), jnp.float32)
    iota16 = lax.iota(jnp.int32, 16)
    base_t = t * EPT

    @pl.loop(0, CPT)
    def _hist(j):
        base = base_t + j * CH
        for g in range(CH // 16):
            msk = (base + g * 16 + iota16) < N_EDGES
            plsc.addupdate_scatter(
                cnt_s, [sidx[j, pl.ds(g * 16, 16)]], ones16, mask=msk)
            plsc.addupdate_scatter(
                cnt_d, [didx[j, pl.ds(g * 16, 16)]], ones16, mask=msk)

    pltpu.sync_copy(cnt_s, out_hbm.at[0, t])
    pltpu.sync_copy(cnt_d, out_hbm.at[1, t])


def _sc_degrees(src_t, dst_t):
    f = pl.kernel(
        _sc_degrees_body,
        jax.ShapeDtypeStruct((2, NT, NB), jnp.float32),
        mesh=_MESH,
        compiler_params=_SC_PARAMS,
        scratch_types=[
            pltpu.VMEM((CPT, CH), jnp.int32),
            pltpu.VMEM((CPT, CH), jnp.int32),
            pltpu.VMEM((NB,), jnp.float32),
            pltpu.VMEM((NB,), jnp.float32),
        ],
    )
    return f(src_t, dst_t)


def _sc_agg_body(h_hbm, src_hbm, dst_hbm, zrows_hbm, out_hbm,
                 sidx, didx, rows, agg_sh, gsems, ssems):
    c = lax.axis_index("c")
    s = lax.axis_index("s")
    t = c * 16 + s
    pltpu.sync_copy(src_hbm.at[t], sidx)
    pltpu.sync_copy(dst_hbm.at[t], didx)
    pltpu.sync_copy(zrows_hbm, agg_sh.at[pl.ds(s * RPS, RPS)])
    plsc.subcore_barrier()

    def start_gather(k, j):
        pltpu.async_copy(h_hbm.at[sidx.at[j]], rows.at[k], gsems.at[k])

    def wait_gather(k, j):
        pltpu.make_async_copy(h_hbm.at[sidx.at[j]], rows.at[k],
                              gsems.at[k]).wait()

    def start_scatter(k, j):
        pltpu.async_copy(rows.at[k], agg_sh.at[didx.at[j]], ssems.at[k],
                         add=True)

    def wait_scatter(k, j):
        pltpu.make_async_copy(rows.at[k], agg_sh.at[didx.at[j]],
                              ssems.at[k]).wait()

    for k in range(NBUF):
        start_gather(k, k)

    @pl.loop(0, NRND - 1)
    def _round(r):
        base = r * NBUF
        for k in range(NBUF):
            wait_gather(k, base + k)
            start_scatter(k, base + k)
        for k in range(NBUF):
            wait_scatter(k, base + k)
            start_gather(k, base + NBUF + k)

    base = (NRND - 1) * NBUF
    for k in range(NBUF):
        wait_gather(k, base + k)
        start_scatter(k, base + k)
    for k in range(NBUF):
        wait_scatter(k, base + k)

    plsc.subcore_barrier()
    pltpu.sync_copy(agg_sh.at[pl.ds(s * RPS, RPS)],
                    out_hbm.at[c, pl.ds(s * RPS, RPS)])


def _sc_aggregate(h, src_t, dst_t, zrows):
    f = pl.kernel(
        _sc_agg_body,
        jax.ShapeDtypeStruct((2, N_AGG, HID_DIM), jnp.float32),
        mesh=_MESH,
        compiler_params=_SC_PARAMS,
        scratch_types=[
            pltpu.VMEM((CPT, CH), jnp.int32),
            pltpu.VMEM((CPT, CH), jnp.int32),
            pltpu.VMEM((NBUF, CH, HID_DIM), jnp.float32),
            pltpu.VMEM_SHARED((N_AGG, HID_DIM), jnp.float32),
            pltpu.SemaphoreType.DMA((NBUF,)),
            pltpu.SemaphoreType.DMA((NBUF,)),
        ],
    )
    return f(h, src_t, dst_t, zrows)


def _tc_degscale_body(dp_ref, out_ref):
    deg_s = jnp.sum(dp_ref[:NT], axis=0, keepdims=True)
    deg_d = jnp.sum(dp_ref[NT:], axis=0, keepdims=True)
    deg = jnp.concatenate([deg_s, deg_d], axis=0)
    out_ref[...] = lax.rsqrt(jnp.maximum(deg, 1.0))


def _tc_degscale(dpart):
    return pl.pallas_call(
        _tc_degscale_body,
        grid=(1,),
        in_specs=[pl.BlockSpec((2 * NT, NB), lambda i: (0, 0))],
        out_specs=pl.BlockSpec((2, NB), lambda i: (0, 0)),
        out_shape=jax.ShapeDtypeStruct((2, NB), jnp.float32),
    )(dpart.reshape(2 * NT, NB))


def _tc_mm1_body(x_ref, w_ref, o_ref):
    o_ref[...] = jnp.dot(x_ref[...], w_ref[...],
                         preferred_element_type=jnp.float32)


def _tc_mm1(x, W1):
    return pl.pallas_call(
        _tc_mm1_body,
        grid=(NBLK,),
        in_specs=[
            pl.BlockSpec((BLK, IN_DIM), lambda i: (i, 0)),
            pl.BlockSpec((IN_DIM, HID_DIM), lambda i: (0, 0)),
        ],
        out_specs=pl.BlockSpec((BLK, HID_DIM), lambda i: (i, 0)),
        out_shape=jax.ShapeDtypeStruct((N_NODES, HID_DIM), jnp.float32),
    )(x, W1)


def _tc_scale_body(y_ref, s_ref, o_ref):
    o_ref[...] = y_ref[...] * s_ref[...]


def _tc_scale(y, scale_col):
    return pl.pallas_call(
        _tc_scale_body,
        grid=(NBLK,),
        in_specs=[
            pl.BlockSpec((BLK, HID_DIM), lambda i: (i, 0)),
            pl.BlockSpec((BLK, 1), lambda i: (i, 0)),
        ],
        out_specs=pl.BlockSpec((BLK, HID_DIM), lambda i: (i, 0)),
        out_shape=jax.ShapeDtypeStruct((N_NODES, HID_DIM), jnp.float32),
    )(y, scale_col)


def _tc_mid_body(p_ref, din_ref, dout_ref, b1_ref, w2_ref, o_ref):
    h = p_ref[0] + p_ref[1]
    h = h * din_ref[...] + b1_ref[...]
    h = jnp.maximum(h, 0.0) * dout_ref[...]
    o_ref[...] = jnp.dot(h, w2_ref[...], preferred_element_type=jnp.float32)


def _tc_mid(p, din_col, dout_col, b1, W2):
    return pl.pallas_call(
        _tc_mid_body,
        grid=(NBLK,),
        in_specs=[
            pl.BlockSpec((2, BLK, HID_DIM), lambda i: (0, i, 0)),
            pl.BlockSpec((BLK, 1), lambda i: (i, 0)),
            pl.BlockSpec((BLK, 1), lambda i: (i, 0)),
            pl.BlockSpec((1, HID_DIM), lambda i: (0, 0)),
            pl.BlockSpec((HID_DIM, OUT_DIM), lambda i: (0, 0)),
        ],
        out_specs=pl.BlockSpec((BLK, OUT_DIM), lambda i: (i, 0)),
        out_shape=jax.ShapeDtypeStruct((N_NODES, OUT_DIM), jnp.float32),
    )(p, din_col, dout_col, b1.reshape(1, HID_DIM), W2)


def _tc_readout_body(p_ref, din_ref, b2_ref, gid_ref, sum_ref, cnt_ref):
    i = pl.program_id(0)
    h = (p_ref[0] + p_ref[1]) * din_ref[...] + b2_ref[...]
    onehot = (gid_ref[...] ==
              lax.broadcasted_iota(jnp.int32, (BLK, N_GRAPHS), 1)
              ).astype(jnp.float32)
    contrib = lax.dot_general(onehot, h, (((0,), (0,)), ((), ())),
                              preferred_element_type=jnp.float32)
    ccontrib = lax.dot_general(onehot, jnp.ones((BLK, 1), jnp.float32),
                               (((0,), (0,)), ((), ())),
                               preferred_element_type=jnp.float32)

    @pl.when(i == 0)
    def _():
        sum_ref[...] = jnp.zeros_like(sum_ref)
        cnt_ref[...] = jnp.zeros_like(cnt_ref)

    sum_ref[...] += contrib
    cnt_ref[...] += ccontrib

    @pl.when(i == NBLK - 1)
    def _():
        sum_ref[...] = sum_ref[...] / jnp.maximum(cnt_ref[...], 1.0)


def _tc_readout(p, din_col, b2, gid_col):
    return pl.pallas_call(
        _tc_readout_body,
        grid=(NBLK,),
        in_specs=[
            pl.BlockSpec((2, BLK, OUT_DIM), lambda i: (0, i, 0)),
            pl.BlockSpec((BLK, 1), lambda i: (i, 0)),
            pl.BlockSpec((1, OUT_DIM), lambda i: (0, 0)),
            pl.BlockSpec((BLK, 1), lambda i: (i, 0)),
        ],
        out_specs=[
            pl.BlockSpec((N_GRAPHS, OUT_DIM), lambda i: (0, 0)),
            pl.BlockSpec((N_GRAPHS, 1), lambda i: (0, 0)),
        ],
        out_shape=[
            jax.ShapeDtypeStruct((N_GRAPHS, OUT_DIM), jnp.float32),
            jax.ShapeDtypeStruct((N_GRAPHS, 1), jnp.float32),
        ],
    )(p, din_col, b2.reshape(1, OUT_DIM), gid_col)


def kernel(x, edge_index, graph_ids, W1, b1, W2, b2):
    src = edge_index[0]
    dst = edge_index[1]
    pad = E_PAD - N_EDGES
    src_t = jnp.concatenate(
        [src, jnp.zeros((pad,), jnp.int32)]).reshape(NT, CPT, CH)
    dst_t = jnp.concatenate(
        [dst, jnp.full((pad,), N_NODES, jnp.int32)]).reshape(NT, CPT, CH)
    zrows = jnp.zeros((RPS, HID_DIM), jnp.float32)

    dpart = _sc_degrees(src_t, dst_t)
    y = _tc_mm1(x, W1)

    scales = _tc_degscale(dpart)
    dout_col = scales[0, :N_NODES, None]
    din_col = scales[1, :N_NODES, None]

    h1pre = _tc_scale(y, dout_col)
    p1 = _sc_aggregate(h1pre, src_t, dst_t, zrows)
    h2pre = _tc_mid(p1[:, :N_NODES], din_col, dout_col, b1, W2)
    p2 = _sc_aggregate(h2pre, src_t, dst_t, zrows)
    sums, _counts = _tc_readout(p2[:, :N_NODES], din_col, b2,
                                graph_ids[:, None])
    return sums.reshape(N_GRAPHS, 1, OUT_DIM)


# speedup vs baseline: 1.9434x; 1.9434x over previous
import jax
import jax.numpy as jnp
from jax import lax
from jax.experimental import pallas as pl
from jax.experimental.pallas import tpu as pltpu
from jax.experimental.pallas import tpu_sc as plsc

N_NODES = 10000
N_EDGES = 320000
N_GRAPHS = 100
IN_DIM = 128
HID_DIM = 64
OUT_DIM = 64

NT = 32
CH = 128
CPT = 80
EPT = CPT * CH
E_PAD = NT * EPT
NB = 10240
NBUF = 4
NRND = CPT // NBUF
N_AGG = 10112
RPS = N_AGG // 16

BLK = 400
NBLK = N_NODES // BLK

_MESH = plsc.VectorSubcoreMesh(core_axis_name="c", subcore_axis_name="s")

_SC_PARAMS = pltpu.CompilerParams()
if "needs_layout_passes" in pltpu.CompilerParams.__dataclass_fields__:
    import dataclasses as _dataclasses
    _SC_PARAMS = _dataclasses.replace(
        _SC_PARAMS, needs_layout_passes=False, use_tc_tiling_on_sc=False)


def _sc_degrees_body(src_hbm, dst_hbm, out_hbm, sidx, didx, cnt_s, cnt_d):
    c = lax.axis_index("c")
    s = lax.axis_index("s")
    t = c * 16 + s
    pltpu.sync_copy(src_hbm.at[t], sidx)
    pltpu.sync_copy(dst_hbm.at[t], didx)

    zero16 = jnp.zeros((16,), jnp.float32)

    @pl.loop(0, NB // 16)
    def _zero(i):
        cnt_s[pl.ds(i * 16, 16)] = zero16
        cnt_d[pl.ds(i * 16, 16)] = zero16

    ones16 = jnp.ones((16,), jnp.float32)
    iota16 = lax.iota(jnp.int32, 16)
    base_t = t * EPT

    @pl.loop(0, CPT)
    def _hist(j):
        base = base_t + j * CH
        for g in range(CH // 16):
            msk = (base + g * 16 + iota16) < N_EDGES
            plsc.addupdate_scatter(
                cnt_s, [sidx[j, pl.ds(g * 16, 16)]], ones16, mask=msk)
            plsc.addupdate_scatter(
                cnt_d, [didx[j, pl.ds(g * 16, 16)]], ones16, mask=msk)

    pltpu.sync_copy(cnt_s, out_hbm.at[0, t])
    pltpu.sync_copy(cnt_d, out_hbm.at[1, t])


def _sc_degrees(src_t, dst_t):
    f = pl.kernel(
        _sc_degrees_body,
        jax.ShapeDtypeStruct((2, NT, NB), jnp.float32),
        mesh=_MESH,
        compiler_params=_SC_PARAMS,
        scratch_types=[
            pltpu.VMEM((CPT, CH), jnp.int32),
            pltpu.VMEM((CPT, CH), jnp.int32),
            pltpu.VMEM((NB,), jnp.float32),
            pltpu.VMEM((NB,), jnp.float32),
        ],
    )
    return f(src_t, dst_t)


def _sc_agg_body(h_hbm, src_hbm, dst_hbm, zrows_hbm, out_hbm,
                 sidx, didx, rows, agg_sh, gsems, ssems):
    c = lax.axis_index("c")
    s = lax.axis_index("s")
    t = c * 16 + s
    pltpu.sync_copy(src_hbm.at[t], sidx)
    pltpu.sync_copy(dst_hbm.at[t], didx)
    pltpu.sync_copy(zrows_hbm, agg_sh.at[pl.ds(s * RPS, RPS)])
    plsc.subcore_barrier()

    def start_gather(k, j):
        pltpu.async_copy(h_hbm.at[sidx.at[j]], rows.at[k], gsems.at[k])

    def wait_gather(k, j):
        pltpu.make_async_copy(h_hbm.at[sidx.at[j]], rows.at[k],
                              gsems.at[k]).wait()

    def start_scatter(k, j):
        pltpu.async_copy(rows.at[k], agg_sh.at[didx.at[j]], ssems.at[k],
                         add=True)

    def wait_scatter(k, j):
        pltpu.make_async_copy(rows.at[k], agg_sh.at[didx.at[j]],
                              ssems.at[k]).wait()

    for k in range(NBUF):
        start_gather(k, k)

    @pl.loop(0, NRND - 1)
    def _round(r):
        base = r * NBUF
        for k in range(NBUF):
            wait_gather(k, base + k)
            start_scatter(k, base + k)
        for k in range(NBUF):
            wait_scatter(k, base + k)
            start_gather(k, base + NBUF + k)

    base = (NRND - 1) * NBUF
    for k in range(NBUF):
        wait_gather(k, base + k)
        start_scatter(k, base + k)
    for k in range(NBUF):
        wait_scatter(k, base + k)

    plsc.subcore_barrier()
    pltpu.sync_copy(agg_sh.at[pl.ds(s * RPS, RPS)],
                    out_hbm.at[c, pl.ds(s * RPS, RPS)])


def _sc_aggregate(h, src_t, dst_t, zrows):
    f = pl.kernel(
        _sc_agg_body,
        jax.ShapeDtypeStruct((2, N_AGG, HID_DIM), jnp.float32),
        mesh=_MESH,
        compiler_params=_SC_PARAMS,
        scratch_types=[
            pltpu.VMEM((CPT, CH), jnp.int32),
            pltpu.VMEM((CPT, CH), jnp.int32),
            pltpu.VMEM((NBUF, CH, HID_DIM), jnp.float32),
            pltpu.VMEM_SHARED((N_AGG, HID_DIM), jnp.float32),
            pltpu.SemaphoreType.DMA((NBUF,)),
            pltpu.SemaphoreType.DMA((NBUF,)),
        ],
    )
    return f(h, src_t, dst_t, zrows)


def _tc_degscale_body(dp_ref, out_ref):
    deg_s = jnp.sum(dp_ref[:NT], axis=0, keepdims=True)
    deg_d = jnp.sum(dp_ref[NT:], axis=0, keepdims=True)
    deg = jnp.concatenate([deg_s, deg_d], axis=0)
    out_ref[...] = lax.rsqrt(jnp.maximum(deg, 1.0))


def _tc_degscale(dpart):
    return pl.pallas_call(
        _tc_degscale_body,
        grid=(1,),
        in_specs=[pl.BlockSpec((2 * NT, NB), lambda i: (0, 0))],
        out_specs=pl.BlockSpec((2, NB), lambda i: (0, 0)),
        out_shape=jax.ShapeDtypeStruct((2, NB), jnp.float32),
    )(dpart.reshape(2 * NT, NB))


def _tc_mm1_body(x_ref, w_ref, o_ref):
    o_ref[...] = jnp.dot(x_ref[...], w_ref[...],
                         preferred_element_type=jnp.float32)


def _tc_mm1(x, W1):
    return pl.pallas_call(
        _tc_mm1_body,
        grid=(NBLK,),
        in_specs=[
            pl.BlockSpec((BLK, IN_DIM), lambda i: (i, 0)),
            pl.BlockSpec((IN_DIM, HID_DIM), lambda i: (0, 0)),
        ],
        out_specs=pl.BlockSpec((BLK, HID_DIM), lambda i: (i, 0)),
        out_shape=jax.ShapeDtypeStruct((N_NODES, HID_DIM), jnp.float32),
    )(x, W1)


def _tc_scale_body(y_ref, s_ref, o_ref):
    o_ref[...] = y_ref[...] * s_ref[...]


def _tc_scale(y, scale_col):
    return pl.pallas_call(
        _tc_scale_body,
        grid=(NBLK,),
        in_specs=[
            pl.BlockSpec((BLK, HID_DIM), lambda i: (i, 0)),
            pl.BlockSpec((BLK, 1), lambda i: (i, 0)),
        ],
        out_specs=pl.BlockSpec((BLK, HID_DIM), lambda i: (i, 0)),
        out_shape=jax.ShapeDtypeStruct((N_NODES, HID_DIM), jnp.float32),
    )(y, scale_col)


def _tc_mid_body(p_ref, din_ref, dout_ref, b1_ref, w2_ref, o_ref):
    h = p_ref[0] + p_ref[1]
    h = h * din_ref[...] + b1_ref[...]
    h = jnp.maximum(h, 0.0) * dout_ref[...]
    o_ref[...] = jnp.dot(h, w2_ref[...], preferred_element_type=jnp.float32)


def _tc_mid(p, din_col, dout_col, b1, W2):
    return pl.pallas_call(
        _tc_mid_body,
        grid=(NBLK,),
        in_specs=[
            pl.BlockSpec((2, BLK, HID_DIM), lambda i: (0, i, 0)),
            pl.BlockSpec((BLK, 1), lambda i: (i, 0)),
            pl.BlockSpec((BLK, 1), lambda i: (i, 0)),
            pl.BlockSpec((1, HID_DIM), lambda i: (0, 0)),
            pl.BlockSpec((HID_DIM, OUT_DIM), lambda i: (0, 0)),
        ],
        out_specs=pl.BlockSpec((BLK, OUT_DIM), lambda i: (i, 0)),
        out_shape=jax.ShapeDtypeStruct((N_NODES, OUT_DIM), jnp.float32),
    )(p, din_col, dout_col, b1.reshape(1, HID_DIM), W2)


def _tc_readout_body(p_ref, din_ref, b2_ref, gid_ref, sum_ref, cnt_ref):
    i = pl.program_id(0)
    h = (p_ref[0] + p_ref[1]) * din_ref[...] + b2_ref[...]
    onehot = (gid_ref[...] ==
              lax.broadcasted_iota(jnp.int32, (BLK, N_GRAPHS), 1)
              ).astype(jnp.float32)
    contrib = lax.dot_general(onehot, h, (((0,), (0,)), ((), ())),
                              preferred_element_type=jnp.float32)
    ccontrib = lax.dot_general(onehot, jnp.ones((BLK, 1), jnp.float32),
                               (((0,), (0,)), ((), ())),
                               preferred_element_type=jnp.float32)

    @pl.when(i == 0)
    def _():
        sum_ref[...] = jnp.zeros_like(sum_ref)
        cnt_ref[...] = jnp.zeros_like(cnt_ref)

    sum_ref[...] += contrib
    cnt_ref[...] += ccontrib

    @pl.when(i == NBLK - 1)
    def _():
        sum_ref[...] = sum_ref[...] / jnp.maximum(cnt_ref[...], 1.0)


def _tc_readout(p, din_col, b2, gid_col):
    return pl.pallas_call(
        _tc_readout_body,
        grid=(NBLK,),
        in_specs=[
            pl.BlockSpec((2, BLK, OUT_DIM), lambda i: (0, i, 0)),
            pl.BlockSpec((BLK, 1), lambda i: (i, 0)),
            pl.BlockSpec((1, OUT_DIM), lambda i: (0, 0)),
            pl.BlockSpec((BLK, 1), lambda i: (i, 0)),
        ],
        out_specs=[
            pl.BlockSpec((N_GRAPHS, OUT_DIM), lambda i: (0, 0)),
            pl.BlockSpec((N_GRAPHS, 1), lambda i: (0, 0)),
        ],
        out_shape=[
            jax.ShapeDtypeStruct((N_GRAPHS, OUT_DIM), jnp.float32),
            jax.ShapeDtypeStruct((N_GRAPHS, 1), jnp.float32),
        ],
    )(p, din_col, b2.reshape(1, OUT_DIM), gid_col)


def kernel(x, edge_index, graph_ids, W1, b1, W2, b2):
    src = edge_index[0]
    dst = edge_index[1]
    pad = E_PAD - N_EDGES
    pad_idx = jnp.arange(pad, dtype=jnp.int32)
    src_t = jnp.concatenate(
        [src, pad_idx % N_NODES]).reshape(NT, CPT, CH)
    dst_t = jnp.concatenate(
        [dst, N_NODES + pad_idx % (N_AGG - N_NODES)]).reshape(NT, CPT, CH)
    zrows = jnp.zeros((RPS, HID_DIM), jnp.float32)

    dpart = _sc_degrees(src_t, dst_t)
    y = _tc_mm1(x, W1)

    scales = _tc_degscale(dpart)
    dout_col = scales[0, :N_NODES, None]
    din_col = scales[1, :N_NODES, None]

    h1pre = _tc_scale(y, dout_col)
    p1 = _sc_aggregate(h1pre, src_t, dst_t, zrows)
    h2pre = _tc_mid(p1[:, :N_NODES], din_col, dout_col, b1, W2)
    p2 = _sc_aggregate(h2pre, src_t, dst_t, zrows)
    sums, _counts = _tc_readout(p2[:, :N_NODES], din_col, b2,
                                graph_ids[:, None])
    return sums.reshape(N_GRAPHS, 1, OUT_DIM)


# speedup vs baseline: 2.5823x; 1.3288x over previous
import dataclasses as _dataclasses

import jax
import jax.numpy as jnp
from jax import lax
from jax.experimental import pallas as pl
from jax.experimental.pallas import tpu as pltpu
from jax.experimental.pallas import tpu_sc as plsc

N_NODES = 10000
N_EDGES = 320000
N_GRAPHS = 100
IN_DIM = 128
HID_DIM = 64
OUT_DIM = 64

NT = 32
EPT = N_EDGES // NT
CH = 128
CFULL = EPT // CH
TAIL = EPT - CFULL * CH
NBUF = 6
NRND = CFULL // NBUF
NR = 10112
RPS = NR // 16

BLK1 = 2000
BLK2 = 2528

_MESH = plsc.VectorSubcoreMesh(core_axis_name="c", subcore_axis_name="s")

_SC_PARAMS = pltpu.CompilerParams()
if "needs_layout_passes" in pltpu.CompilerParams.__dataclass_fields__:
    _SC_PARAMS = _dataclasses.replace(
        _SC_PARAMS, needs_layout_passes=False, use_tc_tiling_on_sc=False)


def _sc_degrees_body(ei_hbm, out_hbm, sidx, didx, cnt_s, cnt_d):
    c = lax.axis_index("c")
    s = lax.axis_index("s")
    t = c * 16 + s
    base = t * EPT
    pltpu.sync_copy(ei_hbm.at[0, pl.ds(base, EPT)], sidx)
    pltpu.sync_copy(ei_hbm.at[1, pl.ds(base, EPT)], didx)

    zero16 = jnp.zeros((16,), jnp.float32)

    @pl.loop(0, N_NODES // 16)
    def _zero(i):
        cnt_s[pl.ds(i * 16, 16)] = zero16
        cnt_d[pl.ds(i * 16, 16)] = zero16

    ones16 = jnp.ones((16,), jnp.float32)

    @pl.loop(0, EPT // 16)
    def _hist(i):
        plsc.addupdate_scatter(cnt_s, [sidx[pl.ds(i * 16, 16)]], ones16)
        plsc.addupdate_scatter(cnt_d, [didx[pl.ds(i * 16, 16)]], ones16)

    pltpu.sync_copy(cnt_s, out_hbm.at[t])
    pltpu.sync_copy(cnt_d, out_hbm.at[NT + t])


def _sc_degrees(edge_index):
    f = pl.kernel(
        _sc_degrees_body,
        jax.ShapeDtypeStruct((2 * NT, N_NODES), jnp.float32),
        mesh=_MESH,
        compiler_params=_SC_PARAMS,
        scratch_types=[
            pltpu.VMEM((EPT,), jnp.int32),
            pltpu.VMEM((EPT,), jnp.int32),
            pltpu.VMEM((N_NODES,), jnp.float32),
            pltpu.VMEM((N_NODES,), jnp.float32),
        ],
    )
    return f(edge_index)


def _sc_agg_body(h_hbm, ei_hbm, out_hbm, sidx, didx, rows, agg_sh,
                 gsems, ssems, zsem):
    c = lax.axis_index("c")
    s = lax.axis_index("s")
    t = c * 16 + s
    base = t * EPT
    pltpu.sync_copy(ei_hbm.at[0, pl.ds(base, EPT)], sidx)
    pltpu.sync_copy(ei_hbm.at[1, pl.ds(base, EPT)], didx)

    zero16 = jnp.zeros((16,), jnp.float32)

    @pl.loop(0, CH)
    def _zrow(r):
        for q in range(HID_DIM // 16):
            rows[0, r, pl.ds(q * 16, 16)] = zero16

    zbase = s * RPS
    for q in range(4):
        pltpu.async_copy(rows.at[0], agg_sh.at[pl.ds(zbase + q * CH, CH)],
                         zsem)
    pltpu.async_copy(rows.at[0, pl.ds(0, RPS - 4 * CH)],
                     agg_sh.at[pl.ds(zbase + 4 * CH, RPS - 4 * CH)], zsem)
    for q in range(4):
        pltpu.make_async_copy(rows.at[0],
                              agg_sh.at[pl.ds(zbase + q * CH, CH)],
                              zsem).wait()
    pltpu.make_async_copy(rows.at[0, pl.ds(0, RPS - 4 * CH)],
                          agg_sh.at[pl.ds(zbase + 4 * CH, RPS - 4 * CH)],
                          zsem).wait()
    plsc.subcore_barrier()

    def gidx(j):
        return sidx.at[pl.ds(j * CH, CH)]

    def widx(j):
        return didx.at[pl.ds(j * CH, CH)]

    def start_gather(k, j):
        pltpu.async_copy(h_hbm.at[gidx(j)], rows.at[k], gsems.at[k])

    def wait_gather(k, j):
        pltpu.make_async_copy(h_hbm.at[gidx(j)], rows.at[k],
                              gsems.at[k]).wait()

    def start_scatter(k, j):
        pltpu.async_copy(rows.at[k], agg_sh.at[widx(j)], ssems.at[k],
                         add=True)

    def wait_scatter(k, j):
        pltpu.make_async_copy(rows.at[k], agg_sh.at[widx(j)],
                              ssems.at[k]).wait()

    for k in range(NBUF):
        start_gather(k, k)

    @pl.loop(0, NRND - 1)
    def _round(r):
        b = r * NBUF
        for k in range(NBUF):
            wait_gather(k, b + k)
            start_scatter(k, b + k)
        for k in range(NBUF):
            wait_scatter(k, b + k)
            start_gather(k, b + NBUF + k)

    b = (NRND - 1) * NBUF
    for k in range(NBUF):
        wait_gather(k, b + k)
        start_scatter(k, b + k)
    for k in range(NBUF):
        wait_scatter(k, b + k)

    toff = CFULL * CH
    pltpu.sync_copy(h_hbm.at[sidx.at[pl.ds(toff, TAIL)]],
                    rows.at[0, pl.ds(0, TAIL)])
    pltpu.sync_copy(rows.at[0, pl.ds(0, TAIL)],
                    agg_sh.at[didx.at[pl.ds(toff, TAIL)]], add=True)

    plsc.subcore_barrier()
    pltpu.sync_copy(agg_sh.at[pl.ds(s * RPS, RPS)],
                    out_hbm.at[c, pl.ds(s * RPS, RPS)])


def _sc_aggregate(h, edge_index):
    f = pl.kernel(
        _sc_agg_body,
        jax.ShapeDtypeStruct((2, NR, HID_DIM), jnp.float32),
        mesh=_MESH,
        compiler_params=_SC_PARAMS,
        scratch_types=[
            pltpu.VMEM((EPT,), jnp.int32),
            pltpu.VMEM((EPT,), jnp.int32),
            pltpu.VMEM((NBUF, CH, HID_DIM), jnp.float32),
            pltpu.VMEM_SHARED((NR, HID_DIM), jnp.float32),
            pltpu.SemaphoreType.DMA((NBUF,)),
            pltpu.SemaphoreType.DMA((NBUF,)),
            pltpu.SemaphoreType.DMA,
        ],
    )
    return f(h, edge_index)


def _tc_degscale_body(dp_ref, out_ref):
    deg_s = jnp.sum(dp_ref[:NT], axis=0, keepdims=True)
    deg_d = jnp.sum(dp_ref[NT:], axis=0, keepdims=True)
    deg = jnp.concatenate([deg_s, deg_d], axis=0)
    out_ref[...] = lax.rsqrt(jnp.maximum(deg, 1.0))


def _tc_degscale(dpart):
    return pl.pallas_call(
        _tc_degscale_body,
        grid=(1,),
        in_specs=[pl.BlockSpec((2 * NT, N_NODES), lambda i: (0, 0))],
        out_specs=pl.BlockSpec((2, N_NODES), lambda i: (0, 0)),
        out_shape=jax.ShapeDtypeStruct((2, N_NODES), jnp.float32),
    )(dpart)


def _tc_mm1_body(x_ref, dout_ref, w_ref, o_ref):
    o_ref[...] = jnp.dot(x_ref[...], w_ref[...],
                         preferred_element_type=jnp.float32) * dout_ref[...]


def _tc_mm1(x, dout_col, W1):
    return pl.pallas_call(
        _tc_mm1_body,
        grid=(N_NODES // BLK1,),
        in_specs=[
            pl.BlockSpec((BLK1, IN_DIM), lambda i: (i, 0)),
            pl.BlockSpec((BLK1, 1), lambda i: (i, 0)),
            pl.BlockSpec((IN_DIM, HID_DIM), lambda i: (0, 0)),
        ],
        out_specs=pl.BlockSpec((BLK1, HID_DIM), lambda i: (i, 0)),
        out_shape=jax.ShapeDtypeStruct((N_NODES, HID_DIM), jnp.float32),
    )(x, dout_col, W1)


def _tc_mid_body(p_ref, din_ref, dout_ref, b1_ref, w2_ref, o_ref):
    h = p_ref[0] + p_ref[1]
    h = h * din_ref[...] + b1_ref[...]
    h = jnp.maximum(h, 0.0) * dout_ref[...]
    o_ref[...] = jnp.dot(h, w2_ref[...], preferred_element_type=jnp.float32)


def _tc_mid(p, din_pad, dout_pad, b1, W2):
    return pl.pallas_call(
        _tc_mid_body,
        grid=(NR // BLK2,),
        in_specs=[
            pl.BlockSpec((2, BLK2, HID_DIM), lambda i: (0, i, 0)),
            pl.BlockSpec((BLK2, 1), lambda i: (i, 0)),
            pl.BlockSpec((BLK2, 1), lambda i: (i, 0)),
            pl.BlockSpec((1, HID_DIM), lambda i: (0, 0)),
            pl.BlockSpec((HID_DIM, OUT_DIM), lambda i: (0, 0)),
        ],
        out_specs=pl.BlockSpec((BLK2, OUT_DIM), lambda i: (i, 0)),
        out_shape=jax.ShapeDtypeStruct((NR, OUT_DIM), jnp.float32),
    )(p, din_pad, dout_pad, b1.reshape(1, HID_DIM), W2)


def _tc_readout_body(p_ref, din_ref, b2_ref, gid_ref, sum_ref, cnt_ref):
    i = pl.program_id(0)
    h = (p_ref[0] + p_ref[1]) * din_ref[...] + b2_ref[...]
    onehot = (gid_ref[...] ==
              lax.broadcasted_iota(jnp.int32, (BLK2, N_GRAPHS), 1)
              ).astype(jnp.float32)
    contrib = lax.dot_general(onehot, h, (((0,), (0,)), ((), ())),
                              preferred_element_type=jnp.float32)
    ccontrib = lax.dot_general(onehot, jnp.ones((BLK2, 1), jnp.float32),
                               (((0,), (0,)), ((), ())),
                               preferred_element_type=jnp.float32)

    @pl.when(i == 0)
    def _():
        sum_ref[...] = jnp.zeros_like(sum_ref)
        cnt_ref[...] = jnp.zeros_like(cnt_ref)

    sum_ref[...] += contrib
    cnt_ref[...] += ccontrib

    @pl.when(i == NR // BLK2 - 1)
    def _():
        sum_ref[...] = sum_ref[...] / jnp.maximum(cnt_ref[...], 1.0)


def _tc_readout(p, din_pad, b2, gid_pad):
    return pl.pallas_call(
        _tc_readout_body,
        grid=(NR // BLK2,),
        in_specs=[
            pl.BlockSpec((2, BLK2, OUT_DIM), lambda i: (0, i, 0)),
            pl.BlockSpec((BLK2, 1), lambda i: (i, 0)),
            pl.BlockSpec((1, OUT_DIM), lambda i: (0, 0)),
            pl.BlockSpec((BLK2, 1), lambda i: (i, 0)),
        ],
        out_specs=[
            pl.BlockSpec((N_GRAPHS, OUT_DIM), lambda i: (0, 0)),
            pl.BlockSpec((N_GRAPHS, 1), lambda i: (0, 0)),
        ],
        out_shape=[
            jax.ShapeDtypeStruct((N_GRAPHS, OUT_DIM), jnp.float32),
            jax.ShapeDtypeStruct((N_GRAPHS, 1), jnp.float32),
        ],
    )(p, din_pad, b2.reshape(1, OUT_DIM), gid_pad)


def kernel(x, edge_index, graph_ids, W1, b1, W2, b2):
    dpart = _sc_degrees(edge_index)
    scales = _tc_degscale(dpart)

    padr = NR - N_NODES
    dout_col = scales[0][:, None]
    din_pad = jnp.concatenate(
        [scales[1], jnp.ones((padr,), jnp.float32)])[:, None]
    dout_pad = jnp.concatenate(
        [scales[0], jnp.ones((padr,), jnp.float32)])[:, None]
    gid_pad = jnp.concatenate(
        [graph_ids, jnp.full((padr,), -1, jnp.int32)])[:, None]

    h1pre = _tc_mm1(x, dout_col, W1)
    p1 = _sc_aggregate(h1pre, edge_index)
    h2pre = _tc_mid(p1, din_pad, dout_pad, b1, W2)
    p2 = _sc_aggregate(h2pre, edge_index)
    sums, _counts = _tc_readout(p2, din_pad, b2, gid_pad)
    return sums.reshape(N_GRAPHS, 1, OUT_DIM)


# speedup vs baseline: 2.6733x; 1.0353x over previous
import dataclasses as _dataclasses

import jax
import jax.numpy as jnp
from jax import lax
from jax.experimental import pallas as pl
from jax.experimental.pallas import tpu as pltpu
from jax.experimental.pallas import tpu_sc as plsc

N_NODES = 10000
N_EDGES = 320000
N_GRAPHS = 100
IN_DIM = 128
HID_DIM = 64
OUT_DIM = 64

NT = 32
EPT = N_EDGES // NT
CH = 128
CFULL = EPT // CH
TAIL = EPT - CFULL * CH
NBUF = 6
NRND = CFULL // NBUF
NR = 10112
RPS = NR // 16

_MESH = plsc.VectorSubcoreMesh(core_axis_name="c", subcore_axis_name="s")

_SC_PARAMS = pltpu.CompilerParams()
if "needs_layout_passes" in pltpu.CompilerParams.__dataclass_fields__:
    _SC_PARAMS = _dataclasses.replace(
        _SC_PARAMS, needs_layout_passes=False, use_tc_tiling_on_sc=False)


def _sc_degrees_body(ei_hbm, out_hbm, sidx, didx, cnt_s, cnt_d):
    c = lax.axis_index("c")
    s = lax.axis_index("s")
    t = c * 16 + s
    base = t * EPT
    pltpu.sync_copy(ei_hbm.at[0, pl.ds(base, EPT)], sidx)
    pltpu.sync_copy(ei_hbm.at[1, pl.ds(base, EPT)], didx)

    zero16 = jnp.zeros((16,), jnp.float32)

    @pl.loop(0, N_NODES // 16)
    def _zero(i):
        cnt_s[pl.ds(i * 16, 16)] = zero16
        cnt_d[pl.ds(i * 16, 16)] = zero16

    ones16 = jnp.ones((16,), jnp.float32)

    @pl.loop(0, EPT // 16)
    def _hist(i):
        plsc.addupdate_scatter(cnt_s, [sidx[pl.ds(i * 16, 16)]], ones16)
        plsc.addupdate_scatter(cnt_d, [didx[pl.ds(i * 16, 16)]], ones16)

    pltpu.sync_copy(cnt_s, out_hbm.at[t])
    pltpu.sync_copy(cnt_d, out_hbm.at[NT + t])


def _sc_degrees(edge_index):
    f = pl.kernel(
        _sc_degrees_body,
        jax.ShapeDtypeStruct((2 * NT, N_NODES), jnp.float32),
        mesh=_MESH,
        compiler_params=_SC_PARAMS,
        scratch_types=[
            pltpu.VMEM((EPT,), jnp.int32),
            pltpu.VMEM((EPT,), jnp.int32),
            pltpu.VMEM((N_NODES,), jnp.float32),
            pltpu.VMEM((N_NODES,), jnp.float32),
        ],
    )
    return f(edge_index)


def _sc_agg_body(h_hbm, ei_hbm, out0_hbm, out1_hbm, sidx, didx, rows,
                 agg_sh, gsems, ssems, zsem):
    c = lax.axis_index("c")
    s = lax.axis_index("s")
    t = c * 16 + s
    base = t * EPT
    pltpu.sync_copy(ei_hbm.at[0, pl.ds(base, EPT)], sidx)
    pltpu.sync_copy(ei_hbm.at[1, pl.ds(base, EPT)], didx)

    zero16 = jnp.zeros((16,), jnp.float32)

    @pl.loop(0, CH)
    def _zrow(r):
        for q in range(HID_DIM // 16):
            rows[0, r, pl.ds(q * 16, 16)] = zero16

    zbase = s * RPS
    for q in range(4):
        pltpu.async_copy(rows.at[0], agg_sh.at[pl.ds(zbase + q * CH, CH)],
                         zsem)
    pltpu.async_copy(rows.at[0, pl.ds(0, RPS - 4 * CH)],
                     agg_sh.at[pl.ds(zbase + 4 * CH, RPS - 4 * CH)], zsem)
    for q in range(4):
        pltpu.make_async_copy(rows.at[0],
                              agg_sh.at[pl.ds(zbase + q * CH, CH)],
                              zsem).wait()
    pltpu.make_async_copy(rows.at[0, pl.ds(0, RPS - 4 * CH)],
                          agg_sh.at[pl.ds(zbase + 4 * CH, RPS - 4 * CH)],
                          zsem).wait()
    plsc.subcore_barrier()

    def gidx(j):
        return sidx.at[pl.ds(j * CH, CH)]

    def widx(j):
        return didx.at[pl.ds(j * CH, CH)]

    def start_gather(k, j):
        pltpu.async_copy(h_hbm.at[gidx(j)], rows.at[k], gsems.at[k])

    def wait_gather(k, j):
        pltpu.make_async_copy(h_hbm.at[gidx(j)], rows.at[k],
                              gsems.at[k]).wait()

    def start_scatter(k, j):
        pltpu.async_copy(rows.at[k], agg_sh.at[widx(j)], ssems.at[k],
                         add=True)

    def wait_scatter(k, j):
        pltpu.make_async_copy(rows.at[k], agg_sh.at[widx(j)],
                              ssems.at[k]).wait()

    for k in range(NBUF):
        start_gather(k, k)

    @pl.loop(0, NRND - 1)
    def _round(r):
        b = r * NBUF
        for k in range(NBUF):
            wait_gather(k, b + k)
            start_scatter(k, b + k)
        for k in range(NBUF):
            wait_scatter(k, b + k)
            start_gather(k, b + NBUF + k)

    b = (NRND - 1) * NBUF
    for k in range(NBUF):
        wait_gather(k, b + k)
        start_scatter(k, b + k)
    for k in range(NBUF):
        wait_scatter(k, b + k)

    toff = CFULL * CH
    pltpu.sync_copy(h_hbm.at[sidx.at[pl.ds(toff, TAIL)]],
                    rows.at[0, pl.ds(0, TAIL)])
    pltpu.sync_copy(rows.at[0, pl.ds(0, TAIL)],
                    agg_sh.at[didx.at[pl.ds(toff, TAIL)]], add=True)

    plsc.subcore_barrier()

    @pl.when(c == 0)
    def _():
        pltpu.sync_copy(agg_sh.at[pl.ds(s * RPS, RPS)],
                        out0_hbm.at[pl.ds(s * RPS, RPS)])

    @pl.when(c == 1)
    def _():
        pltpu.sync_copy(agg_sh.at[pl.ds(s * RPS, RPS)],
                        out1_hbm.at[pl.ds(s * RPS, RPS)])


def _sc_aggregate(h, edge_index):
    f = pl.kernel(
        _sc_agg_body,
        (jax.ShapeDtypeStruct((NR, HID_DIM), jnp.float32),
         jax.ShapeDtypeStruct((NR, HID_DIM), jnp.float32)),
        mesh=_MESH,
        compiler_params=_SC_PARAMS,
        scratch_types=[
            pltpu.VMEM((EPT,), jnp.int32),
            pltpu.VMEM((EPT,), jnp.int32),
            pltpu.VMEM((NBUF, CH, HID_DIM), jnp.float32),
            pltpu.VMEM_SHARED((NR, HID_DIM), jnp.float32),
            pltpu.SemaphoreType.DMA((NBUF,)),
            pltpu.SemaphoreType.DMA((NBUF,)),
            pltpu.SemaphoreType.DMA,
        ],
    )
    return f(h, edge_index)


def _deg_scales(dp):
    r = lax.broadcasted_iota(jnp.int32, (2 * NT, 2), 0)
    col = lax.broadcasted_iota(jnp.int32, (2 * NT, 2), 1)
    sel = jnp.where((col == 0) == (r < NT), 1.0, 0.0).astype(jnp.float32)
    deg = lax.dot_general(dp, sel, (((0,), (0,)), ((), ())),
                          preferred_element_type=jnp.float32)
    return lax.rsqrt(jnp.maximum(deg, 1.0))


def _tc_mm1_body(x_ref, w_ref, o_ref):
    o_ref[...] = jnp.dot(x_ref[...], w_ref[...],
                         preferred_element_type=jnp.float32)


def _tc_mm1(x, W1):
    return pl.pallas_call(
        _tc_mm1_body,
        grid=(1,),
        in_specs=[
            pl.BlockSpec((N_NODES, IN_DIM), lambda i: (0, 0)),
            pl.BlockSpec((IN_DIM, HID_DIM), lambda i: (0, 0)),
        ],
        out_specs=pl.BlockSpec((N_NODES, HID_DIM), lambda i: (0, 0)),
        out_shape=jax.ShapeDtypeStruct((N_NODES, HID_DIM), jnp.float32),
    )(x, W1)


def _tc_scale1_body(y_ref, dp_ref, o_ref):
    scl = _deg_scales(dp_ref[...])
    o_ref[...] = y_ref[...] * scl[:, 0:1]


def _tc_scale1(y, dpart):
    return pl.pallas_call(
        _tc_scale1_body,
        grid=(1,),
        in_specs=[
            pl.BlockSpec((N_NODES, HID_DIM), lambda i: (0, 0)),
            pl.BlockSpec((2 * NT, N_NODES), lambda i: (0, 0)),
        ],
        out_specs=pl.BlockSpec((N_NODES, HID_DIM), lambda i: (0, 0)),
        out_shape=jax.ShapeDtypeStruct((N_NODES, HID_DIM), jnp.float32),
    )(y, dpart)


def _tc_mid_body(p0_ref, p1_ref, dp_ref, b1_ref, w2_ref, o_ref):
    scl = _deg_scales(dp_ref[...])
    h = p0_ref[...] + p1_ref[...]
    h = h * scl[:, 1:2] + b1_ref[...]
    h = jnp.maximum(h, 0.0) * scl[:, 0:1]
    o_ref[...] = jnp.dot(h, w2_ref[...], preferred_element_type=jnp.float32)


def _tc_mid(p0, p1, dpart, b1, W2):
    return pl.pallas_call(
        _tc_mid_body,
        grid=(1,),
        in_specs=[
            pl.BlockSpec((N_NODES, HID_DIM), lambda i: (0, 0)),
            pl.BlockSpec((N_NODES, HID_DIM), lambda i: (0, 0)),
            pl.BlockSpec((2 * NT, N_NODES), lambda i: (0, 0)),
            pl.BlockSpec((1, HID_DIM), lambda i: (0, 0)),
            pl.BlockSpec((HID_DIM, OUT_DIM), lambda i: (0, 0)),
        ],
        out_specs=pl.BlockSpec((N_NODES, OUT_DIM), lambda i: (0, 0)),
        out_shape=jax.ShapeDtypeStruct((N_NODES, OUT_DIM), jnp.float32),
    )(p0, p1, dpart, b1.reshape(1, HID_DIM), W2)


def _tc_readout_body(p0_ref, p1_ref, dp_ref, b2_ref, gid_ref, sum_ref,
                     cnt_ref):
    scl = _deg_scales(dp_ref[...])
    h = (p0_ref[...] + p1_ref[...]) * scl[:, 1:2] + b2_ref[...]
    onehot = (gid_ref[...] ==
              lax.broadcasted_iota(jnp.int32, (N_NODES, N_GRAPHS), 1)
              ).astype(jnp.float32)
    sums = lax.dot_general(onehot, h, (((0,), (0,)), ((), ())),
                           preferred_element_type=jnp.float32)
    cnts = lax.dot_general(onehot, jnp.ones((N_NODES, 1), jnp.float32),
                           (((0,), (0,)), ((), ())),
                           preferred_element_type=jnp.float32)
    cnt_ref[...] = cnts
    sum_ref[...] = sums / jnp.maximum(cnts, 1.0)


def _tc_readout(p0, p1, dpart, b2, gid_col):
    return pl.pallas_call(
        _tc_readout_body,
        grid=(1,),
        in_specs=[
            pl.BlockSpec((N_NODES, OUT_DIM), lambda i: (0, 0)),
            pl.BlockSpec((N_NODES, OUT_DIM), lambda i: (0, 0)),
            pl.BlockSpec((2 * NT, N_NODES), lambda i: (0, 0)),
            pl.BlockSpec((1, OUT_DIM), lambda i: (0, 0)),
            pl.BlockSpec((N_NODES, 1), lambda i: (0, 0)),
        ],
        out_specs=[
            pl.BlockSpec((N_GRAPHS, OUT_DIM), lambda i: (0, 0)),
            pl.BlockSpec((N_GRAPHS, 1), lambda i: (0, 0)),
        ],
        out_shape=[
            jax.ShapeDtypeStruct((N_GRAPHS, OUT_DIM), jnp.float32),
            jax.ShapeDtypeStruct((N_GRAPHS, 1), jnp.float32),
        ],
    )(p0, p1, dpart, b2.reshape(1, OUT_DIM), gid_col)


def kernel(x, edge_index, graph_ids, W1, b1, W2, b2):
    dpart = _sc_degrees(edge_index)
    y = _tc_mm1(x, W1)
    h1pre = _tc_scale1(y, dpart)
    p0, p1 = _sc_aggregate(h1pre, edge_index)
    h2pre = _tc_mid(p0, p1, dpart, b1, W2)
    q0, q1 = _sc_aggregate(h2pre, edge_index)
    sums, _counts = _tc_readout(q0, q1, dpart, b2, graph_ids[:, None])
    return sums.reshape(N_GRAPHS, 1, OUT_DIM)


# speedup vs baseline: 2.7408x; 1.0252x over previous
import dataclasses as _dataclasses

import jax
import jax.numpy as jnp
from jax import lax
from jax.experimental import pallas as pl
from jax.experimental.pallas import tpu as pltpu
from jax.experimental.pallas import tpu_sc as plsc

N_NODES = 10000
N_EDGES = 320000
N_GRAPHS = 100
IN_DIM = 128
HID_DIM = 64
OUT_DIM = 64

NT = 32
EPT = N_EDGES // NT
CH = 128
CFULL = EPT // CH
TAIL = EPT - CFULL * CH
NBUF = 6
NRND = CFULL // NBUF
NR = 10112
RPS = NR // 16

_MESH = plsc.VectorSubcoreMesh(core_axis_name="c", subcore_axis_name="s")

_SC_PARAMS = pltpu.CompilerParams()
if "needs_layout_passes" in pltpu.CompilerParams.__dataclass_fields__:
    _SC_PARAMS = _dataclasses.replace(
        _SC_PARAMS, needs_layout_passes=False, use_tc_tiling_on_sc=False)


def _sc_degrees_body(ei_hbm, out_hbm, sidx, didx, cnt_s, cnt_d):
    c = lax.axis_index("c")
    s = lax.axis_index("s")
    t = c * 16 + s
    base = t * EPT
    pltpu.sync_copy(ei_hbm.at[0, pl.ds(base, EPT)], sidx)
    pltpu.sync_copy(ei_hbm.at[1, pl.ds(base, EPT)], didx)

    zero16 = jnp.zeros((16,), jnp.float32)

    @pl.loop(0, N_NODES // 16, unroll=8)
    def _zero(i):
        cnt_s[pl.ds(i * 16, 16)] = zero16
        cnt_d[pl.ds(i * 16, 16)] = zero16

    ones16 = jnp.ones((16,), jnp.float32)

    @pl.loop(0, EPT // 16, unroll=5)
    def _hist(i):
        plsc.addupdate_scatter(cnt_s, [sidx[pl.ds(i * 16, 16)]], ones16)
        plsc.addupdate_scatter(cnt_d, [didx[pl.ds(i * 16, 16)]], ones16)

    pltpu.sync_copy(cnt_s, out_hbm.at[t])
    pltpu.sync_copy(cnt_d, out_hbm.at[NT + t])


def _sc_degrees(edge_index):
    f = pl.kernel(
        _sc_degrees_body,
        jax.ShapeDtypeStruct((2 * NT, N_NODES), jnp.float32),
        mesh=_MESH,
        compiler_params=_SC_PARAMS,
        scratch_types=[
            pltpu.VMEM((EPT,), jnp.int32),
            pltpu.VMEM((EPT,), jnp.int32),
            pltpu.VMEM((N_NODES,), jnp.float32),
            pltpu.VMEM((N_NODES,), jnp.float32),
        ],
    )
    return f(edge_index)


def _sc_agg_body(h_hbm, ei_hbm, out0_hbm, out1_hbm, sidx, didx, rows,
                 agg_sh, gsems, ssems, zsem):
    c = lax.axis_index("c")
    s = lax.axis_index("s")
    t = c * 16 + s
    base = t * EPT
    idescs = [
        pltpu.async_copy(ei_hbm.at[0, pl.ds(base, EPT)], sidx, gsems.at[0]),
        pltpu.async_copy(ei_hbm.at[1, pl.ds(base, EPT)], didx, gsems.at[1]),
    ]

    zero16 = jnp.zeros((16,), jnp.float32)

    @pl.loop(0, CH)
    def _zrow(r):
        for q in range(HID_DIM // 16):
            rows[0, r, pl.ds(q * 16, 16)] = zero16

    zbase = s * RPS
    for q in range(4):
        pltpu.async_copy(rows.at[0], agg_sh.at[pl.ds(zbase + q * CH, CH)],
                         zsem)
    pltpu.async_copy(rows.at[0, pl.ds(0, RPS - 4 * CH)],
                     agg_sh.at[pl.ds(zbase + 4 * CH, RPS - 4 * CH)], zsem)
    for q in range(4):
        pltpu.make_async_copy(rows.at[0],
                              agg_sh.at[pl.ds(zbase + q * CH, CH)],
                              zsem).wait()
    pltpu.make_async_copy(rows.at[0, pl.ds(0, RPS - 4 * CH)],
                          agg_sh.at[pl.ds(zbase + 4 * CH, RPS - 4 * CH)],
                          zsem).wait()
    for dsc in idescs:
        dsc.wait()
    plsc.subcore_barrier()

    def gidx(j):
        return sidx.at[pl.ds(j * CH, CH)]

    def widx(j):
        return didx.at[pl.ds(j * CH, CH)]

    def start_gather(k, j):
        pltpu.async_copy(h_hbm.at[gidx(j)], rows.at[k], gsems.at[k])

    def wait_gather(k, j):
        pltpu.make_async_copy(h_hbm.at[gidx(j)], rows.at[k],
                              gsems.at[k]).wait()

    def start_scatter(k, j):
        pltpu.async_copy(rows.at[k], agg_sh.at[widx(j)], ssems.at[k],
                         add=True)

    def wait_scatter(k, j):
        pltpu.make_async_copy(rows.at[k], agg_sh.at[widx(j)],
                              ssems.at[k]).wait()

    for k in range(NBUF):
        start_gather(k, k)

    @pl.loop(0, NRND - 1)
    def _round(r):
        b = r * NBUF
        for k in range(NBUF):
            wait_gather(k, b + k)
            start_scatter(k, b + k)
        for k in range(NBUF):
            wait_scatter(k, b + k)
            start_gather(k, b + NBUF + k)

    b = (NRND - 1) * NBUF
    for k in range(NBUF):
        wait_gather(k, b + k)
        start_scatter(k, b + k)
    for k in range(NBUF):
        wait_scatter(k, b + k)

    toff = CFULL * CH
    pltpu.sync_copy(h_hbm.at[sidx.at[pl.ds(toff, TAIL)]],
                    rows.at[0, pl.ds(0, TAIL)])
    pltpu.sync_copy(rows.at[0, pl.ds(0, TAIL)],
                    agg_sh.at[didx.at[pl.ds(toff, TAIL)]], add=True)

    plsc.subcore_barrier()

    @pl.when(c == 0)
    def _():
        pltpu.sync_copy(agg_sh.at[pl.ds(s * RPS, RPS)],
                        out0_hbm.at[pl.ds(s * RPS, RPS)])

    @pl.when(c == 1)
    def _():
        pltpu.sync_copy(agg_sh.at[pl.ds(s * RPS, RPS)],
                        out1_hbm.at[pl.ds(s * RPS, RPS)])


def _sc_aggregate(h, edge_index):
    f = pl.kernel(
        _sc_agg_body,
        (jax.ShapeDtypeStruct((NR, HID_DIM), jnp.float32),
         jax.ShapeDtypeStruct((NR, HID_DIM), jnp.float32)),
        mesh=_MESH,
        compiler_params=_SC_PARAMS,
        scratch_types=[
            pltpu.VMEM((EPT,), jnp.int32),
            pltpu.VMEM((EPT,), jnp.int32),
            pltpu.VMEM((NBUF, CH, HID_DIM), jnp.float32),
            pltpu.VMEM_SHARED((NR, HID_DIM), jnp.float32),
            pltpu.SemaphoreType.DMA((NBUF,)),
            pltpu.SemaphoreType.DMA((NBUF,)),
            pltpu.SemaphoreType.DMA,
        ],
    )
    return f(h, edge_index)


def _deg_scales(dp):
    r = lax.broadcasted_iota(jnp.int32, (2 * NT, 2), 0)
    col = lax.broadcasted_iota(jnp.int32, (2 * NT, 2), 1)
    sel = jnp.where((col == 0) == (r < NT), 1.0, 0.0).astype(jnp.float32)
    deg = lax.dot_general(dp, sel, (((0,), (0,)), ((), ())),
                          preferred_element_type=jnp.float32)
    return lax.rsqrt(jnp.maximum(deg, 1.0))


def _tc_mm1_body(x_ref, w_ref, o_ref):
    o_ref[...] = jnp.dot(x_ref[...], w_ref[...],
                         preferred_element_type=jnp.float32)


def _tc_mm1(x, W1):
    return pl.pallas_call(
        _tc_mm1_body,
        grid=(1,),
        in_specs=[
            pl.BlockSpec((N_NODES, IN_DIM), lambda i: (0, 0)),
            pl.BlockSpec((IN_DIM, HID_DIM), lambda i: (0, 0)),
        ],
        out_specs=pl.BlockSpec((N_NODES, HID_DIM), lambda i: (0, 0)),
        out_shape=jax.ShapeDtypeStruct((N_NODES, HID_DIM), jnp.float32),
    )(x, W1)


def _tc_scale1_body(y_ref, dp_ref, o_ref):
    scl = _deg_scales(dp_ref[...])
    o_ref[...] = y_ref[...] * scl[:, 0:1]


def _tc_scale1(y, dpart):
    return pl.pallas_call(
        _tc_scale1_body,
        grid=(1,),
        in_specs=[
            pl.BlockSpec((N_NODES, HID_DIM), lambda i: (0, 0)),
            pl.BlockSpec((2 * NT, N_NODES), lambda i: (0, 0)),
        ],
        out_specs=pl.BlockSpec((N_NODES, HID_DIM), lambda i: (0, 0)),
        out_shape=jax.ShapeDtypeStruct((N_NODES, HID_DIM), jnp.float32),
    )(y, dpart)


def _tc_mid_body(p0_ref, p1_ref, dp_ref, b1_ref, w2_ref, o_ref):
    scl = _deg_scales(dp_ref[...])
    h = p0_ref[...] + p1_ref[...]
    h = h * scl[:, 1:2] + b1_ref[...]
    h = jnp.maximum(h, 0.0) * scl[:, 0:1]
    o_ref[...] = jnp.dot(h, w2_ref[...], preferred_element_type=jnp.float32)


def _tc_mid(p0, p1, dpart, b1, W2):
    return pl.pallas_call(
        _tc_mid_body,
        grid=(1,),
        in_specs=[
            pl.BlockSpec((N_NODES, HID_DIM), lambda i: (0, 0)),
            pl.BlockSpec((N_NODES, HID_DIM), lambda i: (0, 0)),
            pl.BlockSpec((2 * NT, N_NODES), lambda i: (0, 0)),
            pl.BlockSpec((1, HID_DIM), lambda i: (0, 0)),
            pl.BlockSpec((HID_DIM, OUT_DIM), lambda i: (0, 0)),
        ],
        out_specs=pl.BlockSpec((N_NODES, OUT_DIM), lambda i: (0, 0)),
        out_shape=jax.ShapeDtypeStruct((N_NODES, OUT_DIM), jnp.float32),
    )(p0, p1, dpart, b1.reshape(1, HID_DIM), W2)


def _tc_readout_body(p0_ref, p1_ref, dp_ref, b2_ref, gid_ref, sum_ref,
                     cnt_ref):
    scl = _deg_scales(dp_ref[...])
    h = (p0_ref[...] + p1_ref[...]) * scl[:, 1:2] + b2_ref[...]
    onehot = (gid_ref[...] ==
              lax.broadcasted_iota(jnp.int32, (N_NODES, N_GRAPHS), 1)
              ).astype(jnp.float32)
    sums = lax.dot_general(onehot, h, (((0,), (0,)), ((), ())),
                           preferred_element_type=jnp.float32)
    cnts = lax.dot_general(onehot, jnp.ones((N_NODES, 1), jnp.float32),
                           (((0,), (0,)), ((), ())),
                           preferred_element_type=jnp.float32)
    cnt_ref[...] = cnts
    sum_ref[...] = sums / jnp.maximum(cnts, 1.0)


def _tc_readout(p0, p1, dpart, b2, gid_col):
    return pl.pallas_call(
        _tc_readout_body,
        grid=(1,),
        in_specs=[
            pl.BlockSpec((N_NODES, OUT_DIM), lambda i: (0, 0)),
            pl.BlockSpec((N_NODES, OUT_DIM), lambda i: (0, 0)),
            pl.BlockSpec((2 * NT, N_NODES), lambda i: (0, 0)),
            pl.BlockSpec((1, OUT_DIM), lambda i: (0, 0)),
            pl.BlockSpec((N_NODES, 1), lambda i: (0, 0)),
        ],
        out_specs=[
            pl.BlockSpec((N_GRAPHS, OUT_DIM), lambda i: (0, 0)),
            pl.BlockSpec((N_GRAPHS, 1), lambda i: (0, 0)),
        ],
        out_shape=[
            jax.ShapeDtypeStruct((N_GRAPHS, OUT_DIM), jnp.float32),
            jax.ShapeDtypeStruct((N_GRAPHS, 1), jnp.float32),
        ],
    )(p0, p1, dpart, b2.reshape(1, OUT_DIM), gid_col)


def kernel(x, edge_index, graph_ids, W1, b1, W2, b2):
    dpart = _sc_degrees(edge_index)
    y = _tc_mm1(x, W1)
    h1pre = _tc_scale1(y, dpart)
    p0, p1 = _sc_aggregate(h1pre, edge_index)
    h2pre = _tc_mid(p0, p1, dpart, b1, W2)
    q0, q1 = _sc_aggregate(h2pre, edge_index)
    sums, _counts = _tc_readout(q0, q1, dpart, b2, graph_ids[:, None])
    return sums.reshape(N_GRAPHS, 1, OUT_DIM)
